# baseline (device time: 128119 ns/iter reference)
import jax
import jax.numpy as jnp
from jax import lax
from jax.experimental import pallas as pl
from jax.experimental.pallas import tpu as pltpu

N_DEV = 4
B = 2
SQ = 512
SKV = 512
HG = 2048
HL = 512
NH = 8
DH = 64
DM = 768
ROWS = B * SQ
BLK = ROWS // N_DEV
GW = 32
BAND = 640
NEG = -1e9


def kernel(x, Wq, K_ext, V_ext, Wo):
    xb = x.reshape(ROWS, DM).astype(jnp.bfloat16)
    wqb = Wq.astype(jnp.bfloat16)
    wob = Wo.astype(jnp.bfloat16)
    k2 = K_ext.reshape(B, SKV, HG)
    v2 = V_ext.reshape(B, SKV, HG)

    def body(x_ref, wq_ref, k_ref, v_ref, wo_ref, out_ref,
             kf, vf, kst, vst, kg, vg, qb, cb, pb, pbb, rsb, ags, agb,
             lsem, ksend, vsend, krecv, vrecv, rss, rsr, agss, agr):
        my = lax.axis_index("i")

        ck = pltpu.make_async_copy(k_ref, kf, lsem.at[0])
        cv = pltpu.make_async_copy(v_ref, vf, lsem.at[1])
        ck.start()
        cv.start()

        bsem = pltpu.get_barrier_semaphore()
        for d in range(1, N_DEV):
            pl.semaphore_signal(
                bsem, inc=1,
                device_id=(lax.rem(my + d, N_DEV),),
                device_id_type=pl.DeviceIdType.MESH)
        pl.semaphore_wait(bsem, N_DEV - 1)

        ck.wait()
        cv.wait()
        kst[...] = kf[...].astype(jnp.bfloat16)
        vst[...] = vf[...].astype(jnp.bfloat16)

        sends = []
        for d in range(1, N_DEV):
            tgt = lax.rem(my + d, N_DEV)
            rk = pltpu.make_async_remote_copy(
                src_ref=kst.at[:, :, pl.ds(tgt * HL, HL)],
                dst_ref=kg.at[my],
                send_sem=ksend.at[tgt], recv_sem=krecv.at[my],
                device_id=(tgt,), device_id_type=pl.DeviceIdType.MESH)
            rv = pltpu.make_async_remote_copy(
                src_ref=vst.at[:, :, pl.ds(tgt * HL, HL)],
                dst_ref=vg.at[my],
                send_sem=vsend.at[tgt], recv_sem=vrecv.at[my],
                device_id=(tgt,), device_id_type=pl.DeviceIdType.MESH)
            rk.start()
            rv.start()
            sends += [rk, rv]

        kg[my] = kst[:, :, pl.ds(my * HL, HL)]
        vg[my] = vst[:, :, pl.ds(my * HL, HL)]

        qb[...] = jnp.dot(x_ref[...], wq_ref[...],
                          preferred_element_type=jnp.float32
                          ).astype(jnp.bfloat16)

        def wait_chunk(c):
            @pl.when(c != my)
            def _():
                for (gref, ssem, rsem) in ((kg, ksend, krecv),
                                           (vg, vsend, vrecv)):
                    pltpu.make_async_remote_copy(
                        src_ref=kst.at[:, :, pl.ds(0, HL)],
                        dst_ref=gref.at[c],
                        send_sem=ssem.at[c], recv_sem=rsem.at[c],
                        device_id=(c,),
                        device_id_type=pl.DeviceIdType.MESH).wait_recv()

        wait_chunk(0)
        wait_chunk(1)

        qi = lax.broadcasted_iota(jnp.int32, (SQ, BAND), 0)
        ki = lax.broadcasted_iota(jnp.int32, (SQ, BAND), 1)
        mask = (jnp.abs(qi - ki) <= 128) | (ki < GW) | (qi < GW)

        for b in range(B):
            for h in range(NH):
                cols = pl.ds(h * DH, DH)
                rows = pl.ds(b * SQ, SQ)
                q = qb[rows, cols]
                k = kg[0:2, b, :, cols].reshape(2 * SKV, DH)[0:BAND]
                v = vg[0:2, b, :, cols].reshape(2 * SKV, DH)[0:BAND]
                s = lax.dot_general(
                    q, k, (((1,), (1,)), ((), ())),
                    preferred_element_type=jnp.float32) * 0.125
                s = jnp.where(mask, s, NEG)
                m = jnp.max(s, axis=1, keepdims=True)
                w = jnp.exp(s - m)
                w = (w / jnp.sum(w, axis=1, keepdims=True)).astype(jnp.bfloat16)
                cb[rows, cols] = lax.dot_general(
                    w, v, (((1,), (0,)), ((), ())),
                    preferred_element_type=jnp.float32).astype(jnp.bfloat16)

        for blk in (1, 3):
            r = pl.ds(blk * BLK, BLK)
            pb[r, :] = jnp.dot(cb[r, :], wo_ref[...],
                               preferred_element_type=jnp.float32)
            pbb[r, :] = pb[r, :].astype(jnp.bfloat16)

        rs_sends = []
        for d in range(1, N_DEV):
            tgt = lax.rem(my + d, N_DEV)
            r = pltpu.make_async_remote_copy(
                src_ref=pbb.at[pl.ds(tgt * BLK, BLK)],
                dst_ref=rsb.at[my],
                send_sem=rss.at[tgt], recv_sem=rsr.at[my],
                device_id=(tgt,), device_id_type=pl.DeviceIdType.MESH)
            rs_sends.append((tgt, r))

        wait_chunk(2)
        wait_chunk(3)

        for b in range(B):
            for h in range(NH):
                cols = pl.ds(h * DH, DH)
                rows = pl.ds(b * SQ, GW)
                q = qb[rows, cols]
                k = kg[:, b, :, cols].reshape(N_DEV * SKV, DH)
                v = vg[:, b, :, cols].reshape(N_DEV * SKV, DH)
                s = lax.dot_general(
                    q, k, (((1,), (1,)), ((), ())),
                    preferred_element_type=jnp.float32) * 0.125
                m = jnp.max(s, axis=1, keepdims=True)
                w = jnp.exp(s - m)
                w = (w / jnp.sum(w, axis=1, keepdims=True)).astype(jnp.bfloat16)
                cb[rows, cols] = lax.dot_general(
                    w, v, (((1,), (0,)), ((), ())),
                    preferred_element_type=jnp.float32).astype(jnp.bfloat16)

        for blk in (0, 2):
            r = pl.ds(blk * BLK, BLK)
            pb[r, :] = jnp.dot(cb[r, :], wo_ref[...],
                               preferred_element_type=jnp.float32)
            pbb[r, :] = pb[r, :].astype(jnp.bfloat16)

        for tgt, r in rs_sends:
            r.start()
            sends.append(r)

        acc = pb[pl.ds(my * BLK, BLK), :]
        for d in range(1, N_DEV):
            src = lax.rem(my + d, N_DEV)
            pltpu.make_async_remote_copy(
                src_ref=pbb.at[pl.ds(0, BLK)], dst_ref=rsb.at[src],
                send_sem=rss.at[src], recv_sem=rsr.at[src],
                device_id=(src,),
                device_id_type=pl.DeviceIdType.MESH).wait_recv()
            acc = acc + rsb[src].astype(jnp.float32)
        out_ref[pl.ds(my * BLK, BLK), :] = acc
        ags[...] = acc.astype(jnp.bfloat16)

        for d in range(1, N_DEV):
            tgt = lax.rem(my + d, N_DEV)
            r = pltpu.make_async_remote_copy(
                src_ref=ags, dst_ref=agb.at[my],
                send_sem=agss.at[tgt], recv_sem=agr.at[my],
                device_id=(tgt,), device_id_type=pl.DeviceIdType.MESH)
            r.start()
            sends.append(r)
        for d in range(1, N_DEV):
            src = lax.rem(my + d, N_DEV)
            pltpu.make_async_remote_copy(
                src_ref=ags, dst_ref=agb.at[src],
                send_sem=agss.at[src], recv_sem=agr.at[src],
                device_id=(src,),
                device_id_type=pl.DeviceIdType.MESH).wait_recv()
            out_ref[pl.ds(src * BLK, BLK), :] = agb[src].astype(jnp.float32)

        for r in sends:
            r.wait_send()

    out = pl.pallas_call(
        body,
        out_shape=jax.ShapeDtypeStruct((ROWS, DM), jnp.float32),
        in_specs=[
            pl.BlockSpec(memory_space=pltpu.VMEM),
            pl.BlockSpec(memory_space=pltpu.VMEM),
            pl.BlockSpec(memory_space=pl.ANY),
            pl.BlockSpec(memory_space=pl.ANY),
            pl.BlockSpec(memory_space=pltpu.VMEM),
        ],
        out_specs=pl.BlockSpec(memory_space=pltpu.VMEM),
        scratch_shapes=[
            pltpu.VMEM((B, SKV, HG), jnp.float32),
            pltpu.VMEM((B, SKV, HG), jnp.float32),
            pltpu.VMEM((B, SKV, HG), jnp.bfloat16),
            pltpu.VMEM((B, SKV, HG), jnp.bfloat16),
            pltpu.VMEM((N_DEV, B, SKV, HL), jnp.bfloat16),
            pltpu.VMEM((N_DEV, B, SKV, HL), jnp.bfloat16),
            pltpu.VMEM((ROWS, HL), jnp.bfloat16),
            pltpu.VMEM((ROWS, HL), jnp.bfloat16),
            pltpu.VMEM((ROWS, DM), jnp.float32),
            pltpu.VMEM((ROWS, DM), jnp.bfloat16),
            pltpu.VMEM((N_DEV, BLK, DM), jnp.bfloat16),
            pltpu.VMEM((BLK, DM), jnp.bfloat16),
            pltpu.VMEM((N_DEV, BLK, DM), jnp.bfloat16),
            pltpu.SemaphoreType.DMA((2,)),
            pltpu.SemaphoreType.DMA((N_DEV,)),
            pltpu.SemaphoreType.DMA((N_DEV,)),
            pltpu.SemaphoreType.DMA((N_DEV,)),
            pltpu.SemaphoreType.DMA((N_DEV,)),
            pltpu.SemaphoreType.DMA((N_DEV,)),
            pltpu.SemaphoreType.DMA((N_DEV,)),
            pltpu.SemaphoreType.DMA((N_DEV,)),
            pltpu.SemaphoreType.DMA((N_DEV,)),
        ],
        compiler_params=pltpu.CompilerParams(
            collective_id=0,
            vmem_limit_bytes=100 * 1024 * 1024,
        ),
    )(xb, wqb, k2, v2, wob)
    return out.reshape(B, SQ, DM)


# device time: 112134 ns/iter; 1.1426x vs baseline; 1.1426x over previous
import jax
import jax.numpy as jnp
from jax import lax
from jax.experimental import pallas as pl
from jax.experimental.pallas import tpu as pltpu

N_DEV = 4
B = 2
SQ = 512
SKV = 512
HG = 2048
HL = 512
NH = 8
DH = 64
DM = 768
ROWS = B * SQ
BLK = ROWS // N_DEV
GW = 32
BAND = 640
NEG = -1e9


def kernel(x, Wq, K_ext, V_ext, Wo):
    xb = x.reshape(ROWS, DM).astype(jnp.bfloat16)
    wqb = Wq.astype(jnp.bfloat16)
    wob = Wo.astype(jnp.bfloat16)
    kb = K_ext.reshape(B, SKV, HG).astype(jnp.bfloat16)
    vb = V_ext.reshape(B, SKV, HG).astype(jnp.bfloat16)

    def body(x_ref, wq_ref, k_ref, v_ref, wo_ref, out_ref,
             kg, vg, qb, cb, cba, pb, pbb, corrp, rsb, crecv, ags, agb,
             lsem, ksend, vsend, krecv, vrecv, rss, rsr, cs, cr, agss, agr):
        my = lax.axis_index("i")

        bsem = pltpu.get_barrier_semaphore()
        for d in range(1, N_DEV):
            pl.semaphore_signal(
                bsem, inc=1,
                device_id=(lax.rem(my + d, N_DEV),),
                device_id_type=pl.DeviceIdType.MESH)
        pl.semaphore_wait(bsem, N_DEV - 1)

        sends = []
        for d in range(1, N_DEV):
            tgt = lax.rem(my + d, N_DEV)
            rk = pltpu.make_async_remote_copy(
                src_ref=k_ref.at[:, :, pl.ds(tgt * HL, HL)],
                dst_ref=kg.at[my],
                send_sem=ksend.at[tgt], recv_sem=krecv.at[my],
                device_id=(tgt,), device_id_type=pl.DeviceIdType.MESH)
            rv = pltpu.make_async_remote_copy(
                src_ref=v_ref.at[:, :, pl.ds(tgt * HL, HL)],
                dst_ref=vg.at[my],
                send_sem=vsend.at[tgt], recv_sem=vrecv.at[my],
                device_id=(tgt,), device_id_type=pl.DeviceIdType.MESH)
            rk.start()
            rv.start()
            sends += [rk, rv]

        ck = pltpu.make_async_copy(
            k_ref.at[:, :, pl.ds(my * HL, HL)], kg.at[my], lsem.at[0])
        cv = pltpu.make_async_copy(
            v_ref.at[:, :, pl.ds(my * HL, HL)], vg.at[my], lsem.at[1])
        ck.start()
        cv.start()

        qb[...] = jnp.dot(x_ref[...], wq_ref[...],
                          preferred_element_type=jnp.float32
                          ).astype(jnp.bfloat16)

        ck.wait()
        cv.wait()

        def wait_chunk(c):
            @pl.when(c != my)
            def _():
                for (gref, ssem, rsem) in ((kg, ksend, krecv),
                                           (vg, vsend, vrecv)):
                    pltpu.make_async_remote_copy(
                        src_ref=k_ref.at[:, :, pl.ds(0, HL)],
                        dst_ref=gref.at[c],
                        send_sem=ssem.at[c], recv_sem=rsem.at[c],
                        device_id=(c,),
                        device_id_type=pl.DeviceIdType.MESH).wait_recv()

        wait_chunk(0)
        wait_chunk(1)

        qi = lax.broadcasted_iota(jnp.int32, (SQ, BAND), 0)
        ki = lax.broadcasted_iota(jnp.int32, (SQ, BAND), 1)
        mask = (jnp.abs(qi - ki) <= 128) | (ki < GW) | (qi < GW)

        for b in range(B):
            for h in range(NH):
                cols = pl.ds(h * DH, DH)
                rows = pl.ds(b * SQ, SQ)
                q = qb[rows, cols]
                k = kg[0:2, b, :, cols].reshape(2 * SKV, DH)[0:BAND]
                v = vg[0:2, b, :, cols].reshape(2 * SKV, DH)[0:BAND]
                s = lax.dot_general(
                    q, k, (((1,), (1,)), ((), ())),
                    preferred_element_type=jnp.float32) * 0.125
                s = jnp.where(mask, s, NEG)
                m = jnp.max(s, axis=1, keepdims=True)
                w = jnp.exp(s - m)
                w = (w / jnp.sum(w, axis=1, keepdims=True)).astype(jnp.bfloat16)
                cb[rows, cols] = lax.dot_general(
                    w, v, (((1,), (0,)), ((), ())),
                    preferred_element_type=jnp.float32).astype(jnp.bfloat16)

        cba[0:GW, :] = cb[0:GW, :]
        cba[GW:2 * GW, :] = cb[SQ:SQ + GW, :]

        pb[...] = jnp.dot(cb[...], wo_ref[...],
                          preferred_element_type=jnp.float32)
        pbb[...] = pb[...].astype(jnp.bfloat16)

        for d in range(1, N_DEV):
            tgt = lax.rem(my + d, N_DEV)
            r = pltpu.make_async_remote_copy(
                src_ref=pbb.at[pl.ds(tgt * BLK, BLK)],
                dst_ref=rsb.at[my],
                send_sem=rss.at[tgt], recv_sem=rsr.at[my],
                device_id=(tgt,), device_id_type=pl.DeviceIdType.MESH)
            r.start()
            sends.append(r)

        wait_chunk(2)
        wait_chunk(3)

        for b in range(B):
            for h in range(NH):
                cols = pl.ds(h * DH, DH)
                rows = pl.ds(b * SQ, GW)
                q = qb[rows, cols]
                k = kg[:, b, :, cols].reshape(N_DEV * SKV, DH)
                v = vg[:, b, :, cols].reshape(N_DEV * SKV, DH)
                s = lax.dot_general(
                    q, k, (((1,), (1,)), ((), ())),
                    preferred_element_type=jnp.float32) * 0.125
                m = jnp.max(s, axis=1, keepdims=True)
                w = jnp.exp(s - m)
                w = (w / jnp.sum(w, axis=1, keepdims=True)).astype(jnp.bfloat16)
                cb[rows, cols] = lax.dot_general(
                    w, v, (((1,), (0,)), ((), ())),
                    preferred_element_type=jnp.float32).astype(jnp.bfloat16)

        dnew0 = (cb[0:GW, :].astype(jnp.float32)
                 - cba[0:GW, :].astype(jnp.float32)).astype(jnp.bfloat16)
        dnew2 = (cb[SQ:SQ + GW, :].astype(jnp.float32)
                 - cba[GW:2 * GW, :].astype(jnp.float32)).astype(jnp.bfloat16)
        corrp[0] = jnp.dot(dnew0, wo_ref[...],
                           preferred_element_type=jnp.float32
                           ).astype(jnp.bfloat16)
        corrp[2] = jnp.dot(dnew2, wo_ref[...],
                           preferred_element_type=jnp.float32
                           ).astype(jnp.bfloat16)
        zeros = jnp.zeros((GW, DM), jnp.bfloat16)
        corrp[1] = zeros
        corrp[3] = zeros

        for d in range(1, N_DEV):
            tgt = lax.rem(my + d, N_DEV)
            r = pltpu.make_async_remote_copy(
                src_ref=corrp.at[tgt], dst_ref=crecv.at[my],
                send_sem=cs.at[tgt], recv_sem=cr.at[my],
                device_id=(tgt,), device_id_type=pl.DeviceIdType.MESH)
            r.start()
            sends.append(r)

        acc = pb[pl.ds(my * BLK, BLK), :]
        for d in range(1, N_DEV):
            src = lax.rem(my + d, N_DEV)
            pltpu.make_async_remote_copy(
                src_ref=pbb.at[pl.ds(0, BLK)], dst_ref=rsb.at[src],
                send_sem=rss.at[src], recv_sem=rsr.at[src],
                device_id=(src,),
                device_id_type=pl.DeviceIdType.MESH).wait_recv()
            acc = acc + rsb[src].astype(jnp.float32)
        corr = corrp[my].astype(jnp.float32)
        for d in range(1, N_DEV):
            src = lax.rem(my + d, N_DEV)
            pltpu.make_async_remote_copy(
                src_ref=corrp.at[0], dst_ref=crecv.at[src],
                send_sem=cs.at[src], recv_sem=cr.at[src],
                device_id=(src,),
                device_id_type=pl.DeviceIdType.MESH).wait_recv()
            corr = corr + crecv[src].astype(jnp.float32)
        acc = jnp.concatenate([acc[0:GW, :] + corr, acc[GW:, :]], axis=0)
        out_ref[pl.ds(my * BLK, BLK), :] = acc
        ags[...] = acc.astype(jnp.bfloat16)

        for d in range(1, N_DEV):
            tgt = lax.rem(my + d, N_DEV)
            r = pltpu.make_async_remote_copy(
                src_ref=ags, dst_ref=agb.at[my],
                send_sem=agss.at[tgt], recv_sem=agr.at[my],
                device_id=(tgt,), device_id_type=pl.DeviceIdType.MESH)
            r.start()
            sends.append(r)
        for d in range(1, N_DEV):
            src = lax.rem(my + d, N_DEV)
            pltpu.make_async_remote_copy(
                src_ref=ags, dst_ref=agb.at[src],
                send_sem=agss.at[src], recv_sem=agr.at[src],
                device_id=(src,),
                device_id_type=pl.DeviceIdType.MESH).wait_recv()
            out_ref[pl.ds(src * BLK, BLK), :] = agb[src].astype(jnp.float32)

        for r in sends:
            r.wait_send()

    out = pl.pallas_call(
        body,
        out_shape=jax.ShapeDtypeStruct((ROWS, DM), jnp.float32),
        in_specs=[
            pl.BlockSpec(memory_space=pltpu.VMEM),
            pl.BlockSpec(memory_space=pltpu.VMEM),
            pl.BlockSpec(memory_space=pl.ANY),
            pl.BlockSpec(memory_space=pl.ANY),
            pl.BlockSpec(memory_space=pltpu.VMEM),
        ],
        out_specs=pl.BlockSpec(memory_space=pltpu.VMEM),
        scratch_shapes=[
            pltpu.VMEM((N_DEV, B, SKV, HL), jnp.bfloat16),
            pltpu.VMEM((N_DEV, B, SKV, HL), jnp.bfloat16),
            pltpu.VMEM((ROWS, HL), jnp.bfloat16),
            pltpu.VMEM((ROWS, HL), jnp.bfloat16),
            pltpu.VMEM((2 * GW, HL), jnp.bfloat16),
            pltpu.VMEM((ROWS, DM), jnp.float32),
            pltpu.VMEM((ROWS, DM), jnp.bfloat16),
            pltpu.VMEM((N_DEV, GW, DM), jnp.bfloat16),
            pltpu.VMEM((N_DEV, BLK, DM), jnp.bfloat16),
            pltpu.VMEM((N_DEV, GW, DM), jnp.bfloat16),
            pltpu.VMEM((BLK, DM), jnp.bfloat16),
            pltpu.VMEM((N_DEV, BLK, DM), jnp.bfloat16),
            pltpu.SemaphoreType.DMA((2,)),
            pltpu.SemaphoreType.DMA((N_DEV,)),
            pltpu.SemaphoreType.DMA((N_DEV,)),
            pltpu.SemaphoreType.DMA((N_DEV,)),
            pltpu.SemaphoreType.DMA((N_DEV,)),
            pltpu.SemaphoreType.DMA((N_DEV,)),
            pltpu.SemaphoreType.DMA((N_DEV,)),
            pltpu.SemaphoreType.DMA((N_DEV,)),
            pltpu.SemaphoreType.DMA((N_DEV,)),
            pltpu.SemaphoreType.DMA((N_DEV,)),
            pltpu.SemaphoreType.DMA((N_DEV,)),
        ],
        compiler_params=pltpu.CompilerParams(
            collective_id=0,
            vmem_limit_bytes=100 * 1024 * 1024,
        ),
    )(xb, wqb, kb, vb, wob)
    return out.reshape(B, SQ, DM)
